# Initial kernel scaffold; baseline (speedup 1.0000x reference)
import jax
import jax.numpy as jnp
from jax.experimental import pallas as pl


def kernel(x, batch, annotations, W1, b1, W2, b2):
    raise NotImplementedError("write your pallas kernel here")


# speedup vs baseline: 4.4154x; 4.4154x over previous
import functools

import jax
import jax.numpy as jnp
from jax import lax
from jax.experimental import pallas as pl
from jax.experimental.pallas import tpu as pltpu
from jax.experimental.pallas import tpu_sc as plsc

N = 320000
D = 128
H = 256
C = 16
NSEG = 1024
NW = 32
SEG_PER_W = NSEG // NW
CH = 512
ACC_ROWS = SEG_PER_W + 2
MPAD = 48


def _select_scalar(vec_groups, idx):
  total = jnp.zeros((), jnp.int32)
  lane = lax.broadcasted_iota(jnp.int32, (16,), 0)
  for g, v in enumerate(vec_groups):
    pos = lane + (g * 16)
    total = total + jnp.sum(jnp.where(pos == idx, v, 0))
  return total


def _seg_sum_body(x_hbm, b_hbm, meta_hbm, out_hbm,
                  xbuf, idbuf, idxbuf, zbuf, acc_sh, mbuf):
  cid = lax.axis_index("c")
  sid = lax.axis_index("s")
  w = sid * 2 + cid

  pltpu.sync_copy(meta_hbm, mbuf)
  mv = mbuf[pl.ds(w, 16)]
  r0 = mv[0]
  r1 = mv[1]
  a0 = (r0 // 8) * 8
  nch = (r1 - a0 + (CH - 1)) // CH

  def zero_body(i, carry):
    for c in range(D // 16):
      zbuf[i, pl.ds(c * 16, 16)] = jnp.zeros((16,), jnp.float32)
    return carry
  lax.fori_loop(0, ACC_ROWS, zero_body, 0)
  pltpu.sync_copy(zbuf, acc_sh.at[sid])

  lane = lax.broadcasted_iota(jnp.int32, (16,), 0)
  lo = w * SEG_PER_W - 1

  def chunk_body(k, carry):
    s_unc = a0 + k * CH
    s = jnp.minimum(s_unc, N - CH)
    shift = s_unc - s
    pltpu.sync_copy(x_hbm.at[pl.ds(s, CH)], xbuf)
    pltpu.sync_copy(b_hbm.at[pl.ds(s, CH)], idbuf)
    for j in range(CH // 128):
      for c in range(8):
        g = j * 8 + c
        ids_v = idbuf[pl.ds(g * 16, 16)]
        pos = lane + (g * 16)
        lid = jnp.minimum(jnp.maximum(ids_v - lo, 0), ACC_ROWS - 1)
        lid = jnp.where(pos >= shift, lid, 0)
        idxbuf[j, pl.ds(c * 16, 16)] = lid
    for j in range(CH // 128):
      pltpu.sync_copy(xbuf.at[pl.ds(j * 128, 128)],
                      acc_sh.at[sid].at[idxbuf.at[j]], add=True)
    return carry

  lax.fori_loop(0, nch, chunk_body, 0)

  pltpu.sync_copy(acc_sh.at[sid].at[pl.ds(1, SEG_PER_W)],
                  out_hbm.at[pl.ds(w * SEG_PER_W, SEG_PER_W)])


def _sc_segment_sum(x, batch32, meta):
  mesh = plsc.VectorSubcoreMesh(core_axis_name="c", subcore_axis_name="s")
  kfn = functools.partial(
      pl.kernel,
      mesh=mesh,
      out_type=jax.ShapeDtypeStruct((NSEG, D), jnp.float32),
      scratch_types=[
          pltpu.VMEM((CH, D), jnp.float32),
          pltpu.VMEM((CH,), jnp.int32),
          pltpu.VMEM((CH // 128, 128), jnp.int32),
          pltpu.VMEM((ACC_ROWS, D), jnp.float32),
          pltpu.VMEM_SHARED((16, ACC_ROWS, D), jnp.float32),
          pltpu.VMEM((MPAD,), jnp.int32),
      ],
  )(_seg_sum_body)
  return kfn(x, batch32, meta)


def _mlp_body(p_ref, w1_ref, b1_ref, w2_ref, b2_ref, o_ref):
  h = jnp.dot(p_ref[...], w1_ref[...], preferred_element_type=jnp.float32)
  h = jnp.maximum(h + b1_ref[...], 0.0)
  o = jnp.dot(h, w2_ref[...], preferred_element_type=jnp.float32)
  o_ref[...] = o + b2_ref[...]


def _mlp(pooled, W1, b1, W2, b2):
  return pl.pallas_call(
      _mlp_body,
      out_shape=jax.ShapeDtypeStruct((NSEG, C), jnp.float32),
  )(pooled, W1, b1.reshape(1, H), W2, b2.reshape(1, C))


@jax.jit
def kernel(x, batch, annotations, W1, b1, W2, b2):
  del annotations
  batch32 = batch.astype(jnp.int32)
  qs = jnp.arange(0, NSEG + 1, SEG_PER_W, dtype=jnp.int32)
  bounds = jnp.searchsorted(batch32, qs, side="left").astype(jnp.int32)
  meta = jnp.concatenate(
      [bounds, jnp.full((MPAD - bounds.shape[0],), N, jnp.int32)])
  pooled = _sc_segment_sum(x, batch32, meta)
  return _mlp(pooled, W1, b1, W2, b2)
